# baseline (speedup 1.0000x reference)
import functools

import jax
import jax.numpy as jnp
from jax import lax
from jax.experimental import pallas as pl
from jax.experimental.pallas import tpu as pltpu
from jax.experimental.pallas import tpu_sc as plsc

VOCAB_CHAR = 1000
VOCAB_SEG = 4
CHAR_DIM = 64
SEG_DIM = 16
OUT_DIM = CHAR_DIM + SEG_DIM
FUSED_ROWS = VOCAB_CHAR * VOCAB_SEG

B_ROWS, L = 4096, 200
N = B_ROWS * L
NC, NS = 2, 16
NW = NC * NS
PER_W = N // NW
IDX_W = 128
K = 4
T = K * IDX_W
STEPS = PER_W // T
NBUF = 2
PAIRS = STEPS // NBUF
ROWS_PER_TILE = FUSED_ROWS // NS
LANES = 16

A_ROWS_W = B_ROWS // NW
A_CHUNK = 64
A_CHUNKS = A_ROWS_W // A_CHUNK
A_OFFS = tuple(range(0, L - LANES + 1, LANES)) + (L - LANES,)


def _sc_pack_ids(char_ids, seg_ids):
    mesh = plsc.VectorSubcoreMesh(core_axis_name="c", subcore_axis_name="s")

    @functools.partial(
        pl.kernel,
        out_type=jax.ShapeDtypeStruct((N,), jnp.int32),
        mesh=mesh,
        scratch_types=[
            pltpu.VMEM((A_CHUNK, L), jnp.int32),
            pltpu.VMEM((A_CHUNK, L), jnp.int32),
            pltpu.VMEM((A_CHUNK * L,), jnp.int32),
        ],
    )
    def ka(cid_hbm, sid_hbm, fid_hbm, cbuf, sbuf, obuf):
        wid = lax.axis_index("s") * NC + lax.axis_index("c")
        for chunk in range(A_CHUNKS):
            r0 = wid * A_ROWS_W + chunk * A_CHUNK
            pltpu.sync_copy(cid_hbm.at[pl.ds(r0, A_CHUNK)], cbuf)
            pltpu.sync_copy(sid_hbm.at[pl.ds(r0, A_CHUNK)], sbuf)

            def row_body(r, carry):
                for off in A_OFFS:
                    sl = pl.ds(off, LANES)
                    cv = cbuf[r, sl]
                    sv = sbuf[r, sl]
                    obuf[pl.ds(r * L + off, LANES)] = (cv << 2) + sv
                return carry

            lax.fori_loop(0, A_CHUNK, row_body, 0)
            pltpu.sync_copy(obuf, fid_hbm.at[pl.ds(r0 * L, A_CHUNK * L)])

    return ka(char_ids, seg_ids)


def _sc_gather(fid, char_table, seg_table):
    mesh = plsc.VectorSubcoreMesh(core_axis_name="c", subcore_axis_name="s")

    @functools.partial(
        pl.kernel,
        out_type=jax.ShapeDtypeStruct((N, OUT_DIM), jnp.float32),
        mesh=mesh,
        compiler_params=pltpu.CompilerParams(use_tc_tiling_on_sc=False),
        scratch_types=[
            pltpu.VMEM_SHARED((FUSED_ROWS, OUT_DIM), jnp.float32),
            pltpu.VMEM((NBUF * T,), jnp.int32),
            pltpu.VMEM((2, IDX_W), jnp.int32),
            pltpu.VMEM((NBUF * T, OUT_DIM), jnp.float32),
            pltpu.VMEM((2 * IDX_W, CHAR_DIM), jnp.float32),
            pltpu.VMEM((VOCAB_SEG, SEG_DIM), jnp.float32),
            pltpu.SemaphoreType.DMA,
            pltpu.SemaphoreType.DMA,
            pltpu.SemaphoreType.DMA,
            pltpu.SemaphoreType.DMA,
        ],
    )
    def kb(fid_hbm, ctab_hbm, stab_hbm, out_hbm,
           ftab, fidx, bidx, frows, ctmp, stab_v,
           gsem0, gsem1, wsem0, wsem1):
        sid = lax.axis_index("s")
        wid = sid * NC + lax.axis_index("c")
        iot = lax.iota(jnp.int32, LANES)

        r0 = sid * ROWS_PER_TILE
        for j in range(2):
            for l in range(IDX_W // LANES):
                rvec = r0 + (j * IDX_W + l * LANES) + iot
                cvec = jnp.minimum(rvec >> 2, VOCAB_CHAR - 1)
                bidx[j, pl.ds(l * LANES, LANES)] = cvec
        pltpu.sync_copy(stab_hbm, stab_v)
        g0 = pltpu.async_copy(ctab_hbm.at[bidx.at[0]],
                              ctmp.at[pl.ds(0, IDX_W)], gsem0)
        g1 = pltpu.async_copy(ctab_hbm.at[bidx.at[1]],
                              ctmp.at[pl.ds(IDX_W, IDX_W)], gsem0)
        g0.wait()
        g1.wait()

        def asm_body(i, carry):
            for c4 in range(CHAR_DIM // LANES):
                frows[i, pl.ds(c4 * LANES, LANES)] = (
                    ctmp[i, pl.ds(c4 * LANES, LANES)])
            s = (r0 + i) & (VOCAB_SEG - 1)
            frows[i, pl.ds(CHAR_DIM, SEG_DIM)] = stab_v[s, pl.ds(0, SEG_DIM)]
            return carry

        lax.fori_loop(0, ROWS_PER_TILE, asm_body, 0)
        pltpu.sync_copy(frows.at[pl.ds(0, ROWS_PER_TILE)],
                        ftab.at[pl.ds(r0, ROWS_PER_TILE)])
        plsc.subcore_barrier()

        gsems = (gsem0, gsem1)
        wsems = (wsem0, wsem1)

        def pair_body(p, carry):
            gathers = []
            for buf in range(NBUF):
                i = p * NBUF + buf

                @pl.when(p > 0)
                def _drain():
                    prev_base = wid * PER_W + (i - NBUF) * T
                    pltpu.make_async_copy(
                        frows.at[pl.ds(buf * T, T)],
                        out_hbm.at[pl.ds(prev_base, T)],
                        wsems[buf]).wait()

                base = wid * PER_W + i * T
                pltpu.sync_copy(fid_hbm.at[pl.ds(base, T)],
                                fidx.at[pl.ds(buf * T, T)])
                bg = []
                for j in range(K):
                    bg.append(pltpu.async_copy(
                        ftab.at[fidx.at[pl.ds(buf * T + j * IDX_W, IDX_W)]],
                        frows.at[pl.ds(buf * T + j * IDX_W, IDX_W)],
                        gsems[buf]))
                gathers.append(bg)
            for buf in range(NBUF):
                i = p * NBUF + buf
                for g in gathers[buf]:
                    g.wait()
                base = wid * PER_W + i * T
                pltpu.async_copy(frows.at[pl.ds(buf * T, T)],
                                 out_hbm.at[pl.ds(base, T)], wsems[buf])
            return carry

        lax.fori_loop(0, PAIRS, pair_body, 0)
        for buf in range(NBUF):
            base = wid * PER_W + (STEPS - NBUF + buf) * T
            pltpu.make_async_copy(frows.at[pl.ds(buf * T, T)],
                                  out_hbm.at[pl.ds(base, T)],
                                  wsems[buf]).wait()

    return kb(fid, char_table, seg_table)


def kernel(char_ids, seg_ids, char_table, seg_table):
    fid = _sc_pack_ids(char_ids, seg_ids)
    out = _sc_gather(fid, char_table, seg_table)
    return out.reshape(B_ROWS, L, OUT_DIM)


# speedup vs baseline: 1.0093x; 1.0093x over previous
import functools

import jax
import jax.numpy as jnp
from jax import lax
from jax.experimental import pallas as pl
from jax.experimental.pallas import tpu as pltpu
from jax.experimental.pallas import tpu_sc as plsc

VOCAB_CHAR = 1000
VOCAB_SEG = 4
CHAR_DIM = 64
SEG_DIM = 16
OUT_DIM = CHAR_DIM + SEG_DIM
FUSED_ROWS = VOCAB_CHAR * VOCAB_SEG

B_ROWS, L = 4096, 200
N = B_ROWS * L
NC, NS = 2, 16
NW = NC * NS
BAND = B_ROWS // NW
IDX_W = 128
SPLITS = (104, 96)
NBUF = 2
PAIRS = BAND // NBUF
ROWS_PER_TILE = FUSED_ROWS // NS
LANES = 16

A_CHUNK = 64
A_CHUNKS = BAND // A_CHUNK
A_OFFS = tuple(range(0, L - LANES + 1, LANES)) + (L - LANES,)


def _sc_pack_ids(char_ids, seg_ids):
    mesh = plsc.VectorSubcoreMesh(core_axis_name="c", subcore_axis_name="s")

    @functools.partial(
        pl.kernel,
        out_type=jax.ShapeDtypeStruct((N,), jnp.int32),
        mesh=mesh,
        scratch_types=[
            pltpu.VMEM((A_CHUNK, L), jnp.int32),
            pltpu.VMEM((A_CHUNK, L), jnp.int32),
            pltpu.VMEM((A_CHUNK * L,), jnp.int32),
        ],
    )
    def ka(cid_hbm, sid_hbm, fid_hbm, cbuf, sbuf, obuf):
        wid = lax.axis_index("s") * NC + lax.axis_index("c")
        for chunk in range(A_CHUNKS):
            r0 = wid * BAND + chunk * A_CHUNK
            pltpu.sync_copy(cid_hbm.at[pl.ds(r0, A_CHUNK)], cbuf)
            pltpu.sync_copy(sid_hbm.at[pl.ds(r0, A_CHUNK)], sbuf)

            def row_body(r, carry):
                for off in A_OFFS:
                    sl = pl.ds(off, LANES)
                    cv = cbuf[r, sl]
                    sv = sbuf[r, sl]
                    obuf[pl.ds(r * L + off, LANES)] = (cv << 2) + sv
                return carry

            lax.fori_loop(0, A_CHUNK, row_body, 0)
            pltpu.sync_copy(obuf, fid_hbm.at[pl.ds(r0 * L, A_CHUNK * L)])

    return ka(char_ids, seg_ids)


def _sc_gather(fid, char_table, seg_table):
    mesh = plsc.VectorSubcoreMesh(core_axis_name="c", subcore_axis_name="s")

    @functools.partial(
        pl.kernel,
        out_type=jax.ShapeDtypeStruct((B_ROWS, L, OUT_DIM), jnp.float32),
        mesh=mesh,
        compiler_params=pltpu.CompilerParams(use_tc_tiling_on_sc=False),
        scratch_types=[
            pltpu.VMEM_SHARED((FUSED_ROWS, OUT_DIM), jnp.float32),
            pltpu.VMEM((NBUF, L), jnp.int32),
            pltpu.VMEM((2, IDX_W), jnp.int32),
            pltpu.VMEM((NBUF * L, OUT_DIM), jnp.float32),
            pltpu.VMEM((2 * IDX_W, CHAR_DIM), jnp.float32),
            pltpu.VMEM((VOCAB_SEG, SEG_DIM), jnp.float32),
            pltpu.SemaphoreType.DMA,
            pltpu.SemaphoreType.DMA,
            pltpu.SemaphoreType.DMA,
            pltpu.SemaphoreType.DMA,
        ],
    )
    def kb(fid_hbm, ctab_hbm, stab_hbm, out_hbm,
           ftab, fidx, bidx, frows, ctmp, stab_v,
           gsem0, gsem1, wsem0, wsem1):
        sid = lax.axis_index("s")
        wid = sid * NC + lax.axis_index("c")
        iot = lax.iota(jnp.int32, LANES)

        r0 = sid * ROWS_PER_TILE
        for j in range(2):
            for l in range(IDX_W // LANES):
                rvec = r0 + (j * IDX_W + l * LANES) + iot
                cvec = jnp.minimum(rvec >> 2, VOCAB_CHAR - 1)
                bidx[j, pl.ds(l * LANES, LANES)] = cvec
        pltpu.sync_copy(stab_hbm, stab_v)
        g0 = pltpu.async_copy(ctab_hbm.at[bidx.at[0]],
                              ctmp.at[pl.ds(0, IDX_W)], gsem0)
        g1 = pltpu.async_copy(ctab_hbm.at[bidx.at[1]],
                              ctmp.at[pl.ds(IDX_W, IDX_W)], gsem0)
        g0.wait()
        g1.wait()

        def asm_body(i, carry):
            for c4 in range(CHAR_DIM // LANES):
                frows[i, pl.ds(c4 * LANES, LANES)] = (
                    ctmp[i, pl.ds(c4 * LANES, LANES)])
            s = (r0 + i) & (VOCAB_SEG - 1)
            frows[i, pl.ds(CHAR_DIM, SEG_DIM)] = stab_v[s, pl.ds(0, SEG_DIM)]
            return carry

        lax.fori_loop(0, ROWS_PER_TILE, asm_body, 0)
        pltpu.sync_copy(frows.at[pl.ds(0, ROWS_PER_TILE)],
                        ftab.at[pl.ds(r0, ROWS_PER_TILE)])
        plsc.subcore_barrier()

        gsems = (gsem0, gsem1)
        wsems = (wsem0, wsem1)

        def pair_body(p, carry):
            gathers = []
            for buf in range(NBUF):
                b = wid * BAND + p * NBUF + buf

                @pl.when(p > 0)
                def _drain():
                    pltpu.make_async_copy(
                        frows.at[pl.ds(buf * L, L)],
                        out_hbm.at[b - NBUF],
                        wsems[buf]).wait()

                pltpu.sync_copy(fid_hbm.at[pl.ds(b * L, L)], fidx.at[buf])
                bg = []
                off = 0
                for w in SPLITS:
                    bg.append(pltpu.async_copy(
                        ftab.at[fidx.at[buf, pl.ds(off, w)]],
                        frows.at[pl.ds(buf * L + off, w)],
                        gsems[buf]))
                    off += w
                gathers.append(bg)
            for buf in range(NBUF):
                b = wid * BAND + p * NBUF + buf
                for g in gathers[buf]:
                    g.wait()
                pltpu.async_copy(frows.at[pl.ds(buf * L, L)],
                                 out_hbm.at[b], wsems[buf])
            return carry

        lax.fori_loop(0, PAIRS, pair_body, 0)
        for buf in range(NBUF):
            b = wid * BAND + BAND - NBUF + buf
            pltpu.make_async_copy(frows.at[pl.ds(buf * L, L)],
                                  out_hbm.at[b], wsems[buf]).wait()

    return kb(fid, char_table, seg_table)


def kernel(char_ids, seg_ids, char_table, seg_table):
    fid = _sc_pack_ids(char_ids, seg_ids)
    return _sc_gather(fid, char_table, seg_table)
